# baseline (speedup 1.0000x reference)
import functools

import jax
import jax.numpy as jnp
from jax import lax
from jax.experimental import pallas as pl
from jax.experimental.pallas import tpu as pltpu
from jax.experimental.pallas import tpu_sc as plsc

_D = 768
_E = 64
_N = 32768
_BLK = 4096
_NC = 2
_NS = 16
_NW = _NC * _NS
_TPW = _N // _NW
_L = 16


def _matmul_body(x_ref, wt_ref, b_ref, s_ref):
    x = x_ref[...]
    wt = wt_ref[...]
    s = jnp.dot(x, wt, preferred_element_type=jnp.float32)
    s_ref[...] = s + b_ref[...]


def _tc_scores(inputs, wt, brow):
    return pl.pallas_call(
        _matmul_body,
        grid=(_N // _BLK,),
        in_specs=[
            pl.BlockSpec((_BLK, _D), lambda i: (i, 0)),
            pl.BlockSpec((_D, _E), lambda i: (0, 0)),
            pl.BlockSpec((1, _E), lambda i: (0, 0)),
        ],
        out_specs=pl.BlockSpec((_BLK, _E), lambda i: (i, 0)),
        out_shape=jax.ShapeDtypeStruct((_N, _E), jnp.float32),
        compiler_params=pltpu.CompilerParams(
            dimension_semantics=("arbitrary",),
        ),
    )(inputs, wt, brow)


def _sc_body(s_hbm, p_hbm, i_hbm, sbuf, pbuf, ibuf):
    wid = lax.axis_index("s") * _NC + lax.axis_index("c")
    base = wid * _TPW
    pltpu.sync_copy(s_hbm.at[pl.ds(base * _E, _TPW * _E)], sbuf)

    lanes = lax.iota(jnp.int32, _L)
    zero16 = jnp.zeros((_L,), jnp.int32)

    def group(g, carry):
        flat = (g * _L + lanes) * _E
        m1 = plsc.load_gather(sbuf, [flat])
        i1 = zero16
        m2 = jnp.full((_L,), -jnp.inf, jnp.float32)
        i2 = zero16
        for e in range(1, _E):
            v = plsc.load_gather(sbuf, [flat + e])
            gt1 = v > m1
            gt2 = v > m2
            m2 = jnp.where(gt1, m1, jnp.where(gt2, v, m2))
            i2 = jnp.where(gt1, i1, jnp.where(gt2, e, i2))
            m1 = jnp.where(gt1, v, m1)
            i1 = jnp.where(gt1, e, i1)
        e2 = jnp.exp(m2 - m1)
        p1 = 1.0 / (1.0 + e2)
        p2 = 1.0 - p1
        out = (g * _L + lanes) * 2
        plsc.store_scatter(pbuf, [out], p1)
        plsc.store_scatter(pbuf, [out + 1], p2)
        plsc.store_scatter(ibuf, [out], i1)
        plsc.store_scatter(ibuf, [out + 1], i2)
        return carry

    lax.fori_loop(0, _TPW // _L, group, 0)
    pltpu.sync_copy(pbuf, p_hbm.at[pl.ds(base * 2, _TPW * 2)])
    pltpu.sync_copy(ibuf, i_hbm.at[pl.ds(base * 2, _TPW * 2)])


def _sc_topk(scores):
    mesh = plsc.VectorSubcoreMesh(
        core_axis_name="c", subcore_axis_name="s",
        num_cores=_NC, num_subcores=_NS)
    return pl.kernel(
        _sc_body,
        out_type=[
            jax.ShapeDtypeStruct((_N * 2,), jnp.float32),
            jax.ShapeDtypeStruct((_N * 2,), jnp.int32),
        ],
        mesh=mesh,
        compiler_params=pltpu.CompilerParams(needs_layout_passes=False),
        scratch_types=[
            pltpu.VMEM((_TPW * _E,), jnp.float32),
            pltpu.VMEM((_TPW * 2,), jnp.float32),
            pltpu.VMEM((_TPW * 2,), jnp.int32),
        ],
    )(scores.reshape(-1))


def kernel(inputs, W, b):
    wt = W.T
    brow = b.reshape(1, _E)
    scores = _tc_scores(inputs, wt, brow)
    probs, idx = _sc_topk(scores)
    return (probs.reshape(_N, 2), idx.reshape(_N, 2))
